# baseline (speedup 1.0000x reference)
import functools

import jax
import jax.numpy as jnp
from jax import lax
from jax.experimental import pallas as pl
from jax.experimental.pallas import tpu as pltpu
from jax.experimental.pallas import tpu_sc as plsc

_L = 16
_NBUF = 3


def _vrsqrt(s):
    i = lax.bitcast_convert_type(s, jnp.int32)
    i = jnp.int32(0x5F3759DF) - (i >> 1)
    y = lax.bitcast_convert_type(i, jnp.float32)
    half = s * 0.5
    for _ in range(3):
        y = y * (1.5 - half * y * y)
    return y


def _lane_allreduce(acc):
    lane = lax.iota(jnp.int32, _L)
    dnums = lax.GatherDimensionNumbers(
        offset_dims=(), collapsed_slice_dims=(0,), start_index_map=(0,)
    )
    for sh in (8, 4, 2, 1):
        perm = lax.gather(
            acc, (lane ^ sh)[:, None], dnums, slice_sizes=(1,),
            mode=lax.GatherScatterMode.PROMISE_IN_BOUNDS,
        )
        acc = acc + perm
    return acc


@functools.lru_cache(maxsize=None)
def _make_sc_kernel(n_ids, dim, chunk):
    info = plsc.get_sparse_core_info()
    nw = info.num_cores * info.num_subcores
    assert n_ids % nw == 0
    b_per_w = n_ids // nw
    assert b_per_w % chunk == 0 and chunk % 8 == 0
    n_chunks = b_per_w // chunk
    assert n_chunks >= _NBUF
    n_sl = dim // _L
    mesh = plsc.VectorSubcoreMesh(core_axis_name="c", subcore_axis_name="s")

    @functools.partial(
        pl.kernel,
        mesh=mesh,
        out_type=jax.ShapeDtypeStruct((n_ids, dim), jnp.float32),
        scratch_types=[
            pltpu.VMEM((b_per_w,), jnp.int32),
            pltpu.VMEM((_NBUF, chunk, dim), jnp.float32),
            pltpu.VMEM((chunk * _L,), jnp.float32),
            pltpu.VMEM((chunk,), jnp.float32),
            pltpu.SemaphoreType.DMA((_NBUF,)),
            pltpu.SemaphoreType.DMA((_NBUF,)),
        ],
    )
    def k(table_hbm, ids_hbm, out_hbm, ids_v, bufs, accbuf, scales, gsems, ssems):
        wid = lax.axis_index("s") * info.num_cores + lax.axis_index("c")
        base = wid * b_per_w
        pltpu.sync_copy(ids_hbm.at[pl.ds(base, b_per_w)], ids_v)

        def gather_dma(g, b):
            return pltpu.make_async_copy(
                table_hbm.at[ids_v.at[pl.ds(g * chunk, chunk)]],
                bufs.at[b],
                gsems.at[b],
            )

        def scatter_dma(g, b):
            return pltpu.make_async_copy(
                bufs.at[b],
                out_hbm.at[pl.ds(base + g * chunk, chunk)],
                ssems.at[b],
            )

        def sumsq(row):
            accs = [jnp.zeros((_L,), jnp.float32) for _ in range(4)]
            for j in range(n_sl):
                v = row[pl.ds(j * _L, _L)]
                accs[j % 4] = accs[j % 4] + v * v
            return (accs[0] + accs[1]) + (accs[2] + accs[3])

        def scale_row(row, acc):
            scale = _vrsqrt(jnp.maximum(_lane_allreduce(acc), 1e-24))
            for j in range(n_sl):
                sl = pl.ds(j * _L, _L)
                row[sl] = row[sl] * scale

        def normalize(b):
            nun = 4

            def row_body(i2, acc_prev):
                r1 = nun * i2 + 1
                acc = acc_prev
                for u in range(nun):
                    acc_n = sumsq(bufs.at[b, r1 + u])
                    scale_row(bufs.at[b, r1 + u - 1], acc)
                    acc = acc_n
                return acc

            acc0 = sumsq(bufs.at[b, 0])
            acc_last = lax.fori_loop(0, chunk // nun - 1, row_body, acc0)

            acc = acc_last
            for u in range(nun - 1):
                r = chunk - nun + 1 + u
                acc_n = sumsq(bufs.at[b, r])
                scale_row(bufs.at[b, r - 1], acc)
                acc = acc_n
            scale_row(bufs.at[b, chunk - 1], acc)

        gather_dma(0, 0).start()

        def chunk_body(i, _):
            b = i % _NBUF
            nb = (i + 1) % _NBUF

            @pl.when(i + 1 >= _NBUF)
            def _():
                scatter_dma(i + 1 - _NBUF, nb).wait()

            @pl.when(i + 1 < n_chunks)
            def _():
                gather_dma(i + 1, nb).start()

            gather_dma(i, b).wait()
            normalize(b)
            scatter_dma(i, b).start()
            return 0

        lax.fori_loop(0, n_chunks, chunk_body, 0)

        for g in range(n_chunks - (_NBUF - 1), n_chunks):
            scatter_dma(g, g % _NBUF).wait()

    return k


def kernel(ids, embed_weight):
    bsz, seq = ids.shape
    n_tokens, dim = embed_weight.shape
    ids_flat = ids.reshape(-1).astype(jnp.int32)
    out = _make_sc_kernel(bsz * seq, dim, 32)(embed_weight, ids_flat)
    return out.reshape(bsz, seq, dim)


# speedup vs baseline: 1.0304x; 1.0304x over previous
import functools

import jax
import jax.numpy as jnp
from jax import lax
from jax.experimental import pallas as pl
from jax.experimental.pallas import tpu as pltpu
from jax.experimental.pallas import tpu_sc as plsc

_L = 16
_NBUF = 3


def _vrsqrt(s):
    i = lax.bitcast_convert_type(s, jnp.int32)
    i = jnp.int32(0x5F3759DF) - (i >> 1)
    y = lax.bitcast_convert_type(i, jnp.float32)
    half = s * 0.5
    for _ in range(3):
        y = y * (1.5 - half * y * y)
    return y


def _lane_allreduce(acc):
    lane = lax.iota(jnp.int32, _L)
    dnums = lax.GatherDimensionNumbers(
        offset_dims=(), collapsed_slice_dims=(0,), start_index_map=(0,)
    )
    for sh in (8, 4, 2, 1):
        perm = lax.gather(
            acc, (lane ^ sh)[:, None], dnums, slice_sizes=(1,),
            mode=lax.GatherScatterMode.PROMISE_IN_BOUNDS,
        )
        acc = acc + perm
    return acc


@functools.lru_cache(maxsize=None)
def _make_sc_kernel(n_ids, dim, chunk):
    info = plsc.get_sparse_core_info()
    nw = info.num_cores * info.num_subcores
    assert n_ids % nw == 0
    b_per_w = n_ids // nw
    assert b_per_w % chunk == 0 and chunk % 8 == 0
    n_chunks = b_per_w // chunk
    assert n_chunks >= _NBUF
    n_sl = dim // _L
    mesh = plsc.VectorSubcoreMesh(core_axis_name="c", subcore_axis_name="s")

    @functools.partial(
        pl.kernel,
        mesh=mesh,
        out_type=jax.ShapeDtypeStruct((n_ids, dim), jnp.float32),
        scratch_types=[
            pltpu.VMEM((b_per_w,), jnp.int32),
            pltpu.VMEM((_NBUF, chunk, dim), jnp.float32),
            pltpu.VMEM((chunk * _L,), jnp.float32),
            pltpu.VMEM((chunk,), jnp.float32),
            pltpu.SemaphoreType.DMA((_NBUF,)),
            pltpu.SemaphoreType.DMA((_NBUF,)),
        ],
    )
    def k(table_hbm, ids_hbm, out_hbm, ids_v, bufs, accbuf, scales, gsems, ssems):
        wid = lax.axis_index("s") * info.num_cores + lax.axis_index("c")
        base = wid * b_per_w
        pltpu.sync_copy(ids_hbm.at[pl.ds(base, b_per_w)], ids_v)

        def gather_dma(g, b):
            return pltpu.make_async_copy(
                table_hbm.at[ids_v.at[pl.ds(g * chunk, chunk)]],
                bufs.at[b],
                gsems.at[b],
            )

        def scatter_dma(g, b):
            return pltpu.make_async_copy(
                bufs.at[b],
                out_hbm.at[pl.ds(base + g * chunk, chunk)],
                ssems.at[b],
            )

        def sumsq(row):
            na = 8
            accs = [jnp.zeros((_L,), jnp.float32) for _ in range(na)]
            for j in range(n_sl):
                v = row[pl.ds(j * _L, _L)]
                accs[j % na] = accs[j % na] + v * v
            while len(accs) > 1:
                accs = [a + c for a, c in zip(accs[::2], accs[1::2])]
            return accs[0]

        def scale_row(row, acc):
            scale = _vrsqrt(jnp.maximum(_lane_allreduce(acc), 1e-24))
            for j in range(n_sl):
                sl = pl.ds(j * _L, _L)
                row[sl] = row[sl] * scale

        def normalize(b):
            def row_body(i2, acc_prev):
                r1 = 2 * i2 + 1
                acc_a = sumsq(bufs.at[b, r1])
                scale_row(bufs.at[b, r1 - 1], acc_prev)
                acc_b = sumsq(bufs.at[b, r1 + 1])
                scale_row(bufs.at[b, r1], acc_a)
                return acc_b

            acc0 = sumsq(bufs.at[b, 0])
            acc_last = lax.fori_loop(0, chunk // 2 - 1, row_body, acc0)

            acc_a = sumsq(bufs.at[b, chunk - 1])
            scale_row(bufs.at[b, chunk - 2], acc_last)
            scale_row(bufs.at[b, chunk - 1], acc_a)

        gather_dma(0, 0).start()

        def chunk_body(i, _):
            b = i % _NBUF
            nb = (i + 1) % _NBUF

            @pl.when(i + 1 >= _NBUF)
            def _():
                scatter_dma(i + 1 - _NBUF, nb).wait()

            @pl.when(i + 1 < n_chunks)
            def _():
                gather_dma(i + 1, nb).start()

            gather_dma(i, b).wait()
            normalize(b)
            scatter_dma(i, b).start()
            return 0

        lax.fori_loop(0, n_chunks, chunk_body, 0)

        for g in range(n_chunks - (_NBUF - 1), n_chunks):
            scatter_dma(g, g % _NBUF).wait()

    return k


def kernel(ids, embed_weight):
    bsz, seq = ids.shape
    n_tokens, dim = embed_weight.shape
    ids_flat = ids.reshape(-1).astype(jnp.int32)
    out = _make_sc_kernel(bsz * seq, dim, 32)(embed_weight, ids_flat)
    return out.reshape(bsz, seq, dim)


# speedup vs baseline: 1.0383x; 1.0077x over previous
import functools

import jax
import jax.numpy as jnp
from jax import lax
from jax.experimental import pallas as pl
from jax.experimental.pallas import tpu as pltpu
from jax.experimental.pallas import tpu_sc as plsc

_L = 16
_NBUF = 3


def _vrsqrt(s):
    i = lax.bitcast_convert_type(s, jnp.int32)
    i = jnp.int32(0x5F3759DF) - (i >> 1)
    y = lax.bitcast_convert_type(i, jnp.float32)
    half = s * 0.5
    for _ in range(3):
        y = y * (1.5 - half * y * y)
    return y


def _lane_allreduce(acc):
    lane = lax.iota(jnp.int32, _L)
    dnums = lax.GatherDimensionNumbers(
        offset_dims=(), collapsed_slice_dims=(0,), start_index_map=(0,)
    )
    for sh in (8, 4, 2, 1):
        perm = lax.gather(
            acc, (lane ^ sh)[:, None], dnums, slice_sizes=(1,),
            mode=lax.GatherScatterMode.PROMISE_IN_BOUNDS,
        )
        acc = acc + perm
    return acc


@functools.lru_cache(maxsize=None)
def _make_sc_kernel(n_ids, dim, chunk):
    info = plsc.get_sparse_core_info()
    nw = info.num_cores * info.num_subcores
    assert n_ids % nw == 0
    b_per_w = n_ids // nw
    assert b_per_w % chunk == 0 and chunk % 8 == 0
    n_chunks = b_per_w // chunk
    assert n_chunks >= _NBUF
    n_sl = dim // _L
    mesh = plsc.VectorSubcoreMesh(core_axis_name="c", subcore_axis_name="s")

    @functools.partial(
        pl.kernel,
        mesh=mesh,
        out_type=jax.ShapeDtypeStruct((n_ids, dim), jnp.float32),
        scratch_types=[
            pltpu.VMEM((b_per_w,), jnp.int32),
            pltpu.VMEM((_NBUF, chunk, dim), jnp.float32),
            pltpu.SemaphoreType.DMA((_NBUF,)),
            pltpu.SemaphoreType.DMA((_NBUF,)),
        ],
    )
    def k(table_hbm, ids_hbm, out_hbm, ids_v, bufs, gsems, ssems):
        wid = lax.axis_index("s") * info.num_cores + lax.axis_index("c")
        base = wid * b_per_w
        pltpu.sync_copy(ids_hbm.at[pl.ds(base, b_per_w)], ids_v)

        def gather_dma(g, b):
            return pltpu.make_async_copy(
                table_hbm.at[ids_v.at[pl.ds(g * chunk, chunk)]],
                bufs.at[b],
                gsems.at[b],
            )

        def scatter_dma(g, b):
            return pltpu.make_async_copy(
                bufs.at[b],
                out_hbm.at[pl.ds(base + g * chunk, chunk)],
                ssems.at[b],
            )

        def sumsq(row):
            accs = [jnp.zeros((_L,), jnp.float32) for _ in range(4)]
            for j in range(n_sl):
                v = row[pl.ds(j * _L, _L)]
                accs[j % 4] = accs[j % 4] + v * v
            return (accs[0] + accs[1]) + (accs[2] + accs[3])

        def scale_row(row, acc):
            scale = _vrsqrt(jnp.maximum(_lane_allreduce(acc), 1e-24))
            for j in range(n_sl):
                sl = pl.ds(j * _L, _L)
                row[sl] = row[sl] * scale

        def normalize(b):
            def row_body(i2, acc_prev):
                r1 = 2 * i2 + 1
                acc_a = sumsq(bufs.at[b, r1])
                scale_row(bufs.at[b, r1 - 1], acc_prev)
                acc_b = sumsq(bufs.at[b, r1 + 1])
                scale_row(bufs.at[b, r1], acc_a)
                return acc_b

            acc0 = sumsq(bufs.at[b, 0])
            acc_last = lax.fori_loop(0, chunk // 2 - 1, row_body, acc0)

            acc_a = sumsq(bufs.at[b, chunk - 1])
            scale_row(bufs.at[b, chunk - 2], acc_last)
            scale_row(bufs.at[b, chunk - 1], acc_a)

        gather_dma(0, 0).start()

        def chunk_body(i, _):
            b = i % _NBUF
            nb = (i + 1) % _NBUF

            @pl.when(i + 1 >= _NBUF)
            def _():
                scatter_dma(i + 1 - _NBUF, nb).wait()

            @pl.when(i + 1 < n_chunks)
            def _():
                gather_dma(i + 1, nb).start()

            gather_dma(i, b).wait()
            normalize(b)
            scatter_dma(i, b).start()
            return 0

        lax.fori_loop(0, n_chunks, chunk_body, 0)

        for g in range(n_chunks - (_NBUF - 1), n_chunks):
            scatter_dma(g, g % _NBUF).wait()

    return k


def kernel(ids, embed_weight):
    bsz, seq = ids.shape
    n_tokens, dim = embed_weight.shape
    ids_flat = ids.reshape(-1).astype(jnp.int32)
    out = _make_sc_kernel(bsz * seq, dim, 32)(embed_weight, ids_flat)
    return out.reshape(bsz, seq, dim)
